# baseline (speedup 1.0000x reference)
import functools

import jax
import jax.numpy as jnp
from jax import lax
from jax.experimental import pallas as pl
from jax.experimental.pallas import tpu as pltpu

_BLK = 10000
_SEGW = (_BLK // 128 + 1) * 128
_R = _BLK % 128
_P = 128 // _R
_NUM_GRAPHS = 64
_LN2 = 0.6931471805599453
_LOG2E = 1.4426950408889634


def _body(nblk, v_ref, b_ref, w1_ref, b1_ref, w2_ref, b2_ref, out_ref,
          acc_ref, cnt_ref, seg_ref, hp_ref, sem):
    i = pl.program_id(0)

    @pl.when(i == 0)
    def _init():
        acc_ref[...] = jnp.zeros_like(acc_ref)
        cnt_ref[...] = jnp.zeros_like(cnt_ref)

    off = ((_BLK // 128) * i + i // _P) * 128
    m = i * _BLK - off
    cp = pltpu.make_async_copy(b_ref.at[pl.ds(off, _SEGW)], seg_ref.at[0], sem)
    cp.start()

    x = v_ref[...]
    y = lax.dot_general(x, w1_ref[...] * _LOG2E, (((1,), (1,)), ((), ())),
                        preferred_element_type=jnp.float32)
    y = y + b1_ref[...] * _LOG2E
    h = (jnp.maximum(y, 0.0) - 1.0) + jnp.log2(1.0 + jnp.exp2(jnp.minimum(y, -y)))

    hp_ref[0:128, :] = jnp.zeros((128, hp_ref.shape[1]), jnp.float32)
    hp_ref[_BLK:_SEGW, :] = jnp.zeros((_SEGW - _BLK, hp_ref.shape[1]),
                                      jnp.float32)
    hp_ref[pl.ds((i % _P) * _R, _BLK), :] = h

    cp.wait()
    seg = seg_ref[...]
    jj = lax.broadcasted_iota(jnp.int32, (1, _SEGW), 1)
    seg = jnp.where((jj >= m) & (jj < m + _BLK), seg, _NUM_GRAPHS)
    gids = lax.broadcasted_iota(jnp.int32, (_NUM_GRAPHS, _SEGW), 0)
    oh = (gids == seg).astype(jnp.float32)
    acc_ref[...] += jnp.dot(oh, hp_ref[...], preferred_element_type=jnp.float32)
    cnt_ref[...] += jnp.sum(oh, axis=1, keepdims=True)

    out_ref[...] = jnp.zeros_like(out_ref)

    @pl.when(i == nblk - 1)
    def _finish():
        u0 = jnp.dot(acc_ref[...], w2_ref[...],
                     preferred_element_type=jnp.float32) * _LN2
        out_ref[0:_NUM_GRAPHS, :] = u0 + cnt_ref[...] * b2_ref[...]


def kernel(v, batch, W1, b1, W2, b2):
    n, hidden = v.shape
    out_dim = W2.shape[1]
    nblk = n // _BLK
    batch_r = jnp.pad(batch.astype(jnp.int32), (0, _SEGW - _BLK))
    W1t = W1.T
    b1r = b1.reshape(1, -1)
    b2r = b2.reshape(1, -1)
    return pl.pallas_call(
        functools.partial(_body, nblk),
        grid=(nblk,),
        in_specs=[
            pl.BlockSpec((_BLK, hidden), lambda i: (i, 0)),
            pl.BlockSpec(memory_space=pl.ANY),
            pl.BlockSpec(W1t.shape, lambda i: (0, 0)),
            pl.BlockSpec(b1r.shape, lambda i: (0, 0)),
            pl.BlockSpec(W2.shape, lambda i: (0, 0)),
            pl.BlockSpec(b2r.shape, lambda i: (0, 0)),
        ],
        out_specs=pl.BlockSpec((_BLK, out_dim), lambda i: ((i + 1) % nblk, 0)),
        out_shape=jax.ShapeDtypeStruct((n, out_dim), jnp.float32),
        scratch_shapes=[
            pltpu.VMEM((_NUM_GRAPHS, W1.shape[1]), jnp.float32),
            pltpu.VMEM((_NUM_GRAPHS, 1), jnp.float32),
            pltpu.VMEM((1, _SEGW), jnp.int32),
            pltpu.VMEM((_SEGW, W1.shape[1]), jnp.float32),
            pltpu.SemaphoreType.DMA,
        ],
    )(v, batch_r, W1t, b1r, W2, b2r)
